# baseline (speedup 1.0000x reference)
import functools

import jax
import jax.numpy as jnp
from jax import lax
from jax.experimental import pallas as pl
from jax.experimental.pallas import tpu as pltpu
from jax.experimental.pallas import tpu_sc as plsc

_N = 10000
_E = 320000
_G = 64
_DX = 104
_DD = 12
_EMB = 12
_NST = 256
_H = 128
_OUT = 10

_NC = 2
_NS = 16
_NW = _NC * _NS
_EPW = _E // _NW
_RPT = 624
_RREM = _N - _NS * _RPT
_NB = 3

_BN = 1000
_NBLK = _N // _BN


def _sc_agg_body(w, ch, nb, *refs):
    (h_hbm, src_hbm, dst_hbm, zrows_hbm, out_agg,
     acc, sidx, didx, *bufs) = refs
    rows = bufs[0]
    gsems = tuple(bufs[1:1 + nb])
    ssems = tuple(bufs[1 + nb:1 + 2 * nb])
    nch = _EPW // ch
    c = lax.axis_index("c")
    s = lax.axis_index("s")
    wid = s * _NC + c

    ebase = pl.multiple_of(wid * _EPW, 8)
    pltpu.sync_copy(src_hbm.at[pl.ds(ebase, _EPW)], sidx)
    pltpu.sync_copy(dst_hbm.at[pl.ds(ebase, _EPW)], didx)

    row0 = pl.multiple_of(s * _RPT, 8)
    pltpu.sync_copy(zrows_hbm.at[pl.ds(row0, _RPT)],
                    acc.at[pl.ds(row0, _RPT)])

    @pl.when(s == 0)
    def _():
        pltpu.sync_copy(zrows_hbm.at[pl.ds(_NS * _RPT, _RREM)],
                        acc.at[pl.ds(_NS * _RPT, _RREM)])
    plsc.subcore_barrier()

    def _idx(ref, j):
        st = pl.multiple_of(j * ch, 8)
        return ref.at[pl.ds(st, ch)]

    def _fire_g(j, b):
        pltpu.async_copy(h_hbm.at[_idx(sidx, j)], rows.at[b], gsems[b])

    def _wait_g(b):
        pltpu.make_async_copy(h_hbm.at[_idx(sidx, 0)], rows.at[b],
                              gsems[b]).wait()

    def _fire_s(j, b):
        pltpu.async_copy(rows.at[b], acc.at[_idx(didx, j)], ssems[b],
                         add=True)

    def _wait_s(b):
        pltpu.make_async_copy(rows.at[b], acc.at[_idx(didx, 0)],
                              ssems[b]).wait()

    lag = nb - 1
    for b in range(nb):
        _fire_g(b, b)
    for k in range(lag):
        _wait_g(k % nb)
        _fire_s(k, k % nb)
        _wait_s(k % nb)
        _fire_g(k + nb, k % nb)
    for k in range(lag, nb):
        _wait_g(k)
        _fire_s(k, k)

    ngrp = (nch - lag) // nb
    def _group(grp, carry):
        for r in range(nb):
            k = grp * nb + r
            ba = (r + lag) % nb
            _wait_s(ba)
            _fire_g(k + lag, ba)
            _wait_g(r)
            _fire_s(k, r)
        return carry

    lax.fori_loop(1, ngrp, _group, 0)

    for j in range(ngrp * nb, nch):
        if j + lag < nch:
            _wait_s((j + lag) % nb)
            _fire_g(j + lag, (j + lag) % nb)
        _wait_g(j % nb)
        _fire_s(j, j % nb)
    for b in range(nb):
        _wait_s(b)
    plsc.subcore_barrier()

    pltpu.sync_copy(acc.at[pl.ds(row0, _RPT)],
                    out_agg.at[c, pl.ds(row0, _RPT)])

    @pl.when(s == 0)
    def _():
        pltpu.sync_copy(acc.at[pl.ds(_NS * _RPT, _RREM)],
                        out_agg.at[c, pl.ds(_NS * _RPT, _RREM)])


def _sc_mesh():
    return plsc.VectorSubcoreMesh(core_axis_name="c", subcore_axis_name="s",
                                  num_cores=_NC, num_subcores=_NS)


def _make_sc_agg(w, ch, nb):
    return pl.kernel(
        functools.partial(_sc_agg_body, w, ch, nb),
        out_type=jax.ShapeDtypeStruct((_NC, _N, w), jnp.float32),
        mesh=_sc_mesh(),
        scratch_types=[
            pltpu.VMEM_SHARED((_N, w), jnp.float32),
            pltpu.VMEM((_EPW,), jnp.int32),
            pltpu.VMEM((_EPW,), jnp.int32),
            pltpu.VMEM((nb, ch, w), jnp.float32),
        ] + [pltpu.SemaphoreType.DMA] * (2 * nb),
    )


def _sc_cnt_body(dst_hbm, zrows_hbm, out_cnt, cacc, didx, ones, csem):
    c = lax.axis_index("c")
    s = lax.axis_index("s")
    wid = s * _NC + c

    ebase = pl.multiple_of(wid * _EPW, 8)
    pltpu.sync_copy(dst_hbm.at[pl.ds(ebase, _EPW)], didx)
    row0 = pl.multiple_of(s * _RPT, 8)
    pltpu.sync_copy(zrows_hbm.at[pl.ds(row0, _RPT)],
                    cacc.at[pl.ds(row0, _RPT)])

    @pl.when(s == 0)
    def _():
        pltpu.sync_copy(zrows_hbm.at[pl.ds(_NS * _RPT, _RREM)],
                        cacc.at[pl.ds(_NS * _RPT, _RREM)])

    def _fill(k, carry):
        for q in range(_H // 16):
            ones[k, pl.ds(q * 16, 16)] = jnp.full((16,), 1.0, jnp.float32)
        return carry

    lax.fori_loop(0, 80, _fill, 0)
    plsc.subcore_barrier()

    def _scat(j):
        st = pl.multiple_of(j * 80, 8)
        pltpu.async_copy(ones, cacc.at[didx.at[pl.ds(st, 80)]], csem,
                         add=True)

    def _drain():
        pltpu.make_async_copy(ones, cacc.at[didx.at[pl.ds(0, 80)]],
                              csem).wait()

    grp = 5
    def _body(g, carry):
        for b in range(grp):
            _scat(g * grp + b)
        for _ in range(grp):
            _drain()
        return carry

    lax.fori_loop(0, (_EPW // 80) // grp, _body, 0)
    plsc.subcore_barrier()

    pltpu.sync_copy(cacc.at[pl.ds(row0, _RPT)],
                    out_cnt.at[c, pl.ds(row0, _RPT)])

    @pl.when(s == 0)
    def _():
        pltpu.sync_copy(cacc.at[pl.ds(_NS * _RPT, _RREM)],
                        out_cnt.at[c, pl.ds(_NS * _RPT, _RREM)])


def _make_sc_cnt():
    return pl.kernel(
        _sc_cnt_body,
        out_type=jax.ShapeDtypeStruct((_NC, _N, _H), jnp.float32),
        mesh=_sc_mesh(),
        scratch_types=[
            pltpu.VMEM_SHARED((_N, _H), jnp.float32),
            pltpu.VMEM((_EPW,), jnp.int32),
            pltpu.VMEM((80, _H), jnp.float32),
            pltpu.SemaphoreType.DMA,
        ],
    )


@functools.lru_cache(maxsize=None)
def _sc_cached(kind, *args):
    return _make_sc_agg(*args) if kind == "agg" else _make_sc_cnt()


def _embed_body(x_ref, xd_ref, xst_ref, st_ref, out_ref):
    xst = xst_ref[0, 0, :]
    oh = (xst[:, None] == lax.broadcasted_iota(jnp.int32, (_BN, _NST), 1))
    emb = jnp.dot(oh.astype(jnp.float32), st_ref[...],
                  preferred_element_type=jnp.float32)
    out_ref[...] = jnp.concatenate([x_ref[...], xd_ref[...], emb], axis=1)


def _embed_concat(x, xdims, xst3, st_table):
    return pl.pallas_call(
        _embed_body,
        grid=(_NBLK,),
        in_specs=[
            pl.BlockSpec((_BN, _DX), lambda i: (i, 0)),
            pl.BlockSpec((_BN, _DD), lambda i: (i, 0)),
            pl.BlockSpec((1, 1, _BN), lambda i: (i, 0, 0)),
            pl.BlockSpec((_NST, _EMB), lambda i: (0, 0)),
        ],
        out_specs=pl.BlockSpec((_BN, _H), lambda i: (i, 0)),
        out_shape=jax.ShapeDtypeStruct((_N, _H), jnp.float32),
    )(x, xdims, xst3, st_table)


def _layer_math(agg_ref, cnt_ref, h_ref, wl_ref, wr_ref, b_ref, g_ref, be_ref):
    cnt = cnt_ref[0, :, 0:1] + cnt_ref[1, :, 0:1]
    inv = 1.0 / jnp.maximum(cnt, 1.0)
    mean = (agg_ref[0] + agg_ref[1]) * inv
    z = (jnp.dot(mean, wl_ref[...], preferred_element_type=jnp.float32)
         + jnp.dot(h_ref[...], wr_ref[...], preferred_element_type=jnp.float32)
         + b_ref[...])
    m = jnp.mean(z, axis=-1, keepdims=True)
    zc = z - m
    v = jnp.mean(zc * zc, axis=-1, keepdims=True)
    hn = zc * lax.rsqrt(v + 1e-5) * g_ref[...] + be_ref[...]
    return jnp.maximum(hn, 0.0)


def _layer_body(agg_ref, cnt_ref, h_ref, wl_ref, wr_ref, b_ref, g_ref,
                be_ref, out_ref):
    out_ref[...] = _layer_math(agg_ref, cnt_ref, h_ref, wl_ref, wr_ref,
                               b_ref, g_ref, be_ref)


_W_SPECS = [
    pl.BlockSpec((_H, _H), lambda i: (0, 0)),
    pl.BlockSpec((_H, _H), lambda i: (0, 0)),
    pl.BlockSpec((_H,), lambda i: (0,)),
    pl.BlockSpec((_H,), lambda i: (0,)),
    pl.BlockSpec((_H,), lambda i: (0,)),
]


def _layer(agg, cnt3, h, wl, wr, b, g, be):
    return pl.pallas_call(
        _layer_body,
        grid=(_NBLK,),
        in_specs=[
            pl.BlockSpec((_NC, _BN, _H), lambda i: (0, i, 0)),
            pl.BlockSpec((_NC, _BN, _H), lambda i: (0, i, 0)),
            pl.BlockSpec((_BN, _H), lambda i: (i, 0)),
        ] + _W_SPECS,
        out_specs=pl.BlockSpec((_BN, _H), lambda i: (i, 0)),
        out_shape=jax.ShapeDtypeStruct((_N, _H), jnp.float32),
    )(agg, cnt3, h, wl, wr, b, g, be)


def _layer_pool_body(agg_ref, cnt_ref, h_ref, wl_ref, wr_ref, b_ref, g_ref,
                     be_ref, bt_ref, psum_ref, pmax_ref, pcnt_ref):
    i = pl.program_id(0)
    h2 = _layer_math(agg_ref, cnt_ref, h_ref, wl_ref, wr_ref, b_ref, g_ref,
                     be_ref)
    bt = bt_ref[0, 0, :]
    oh = (bt[:, None] == lax.broadcasted_iota(jnp.int32, (_BN, _G), 1))
    ohf = oh.astype(jnp.float32)
    contrib_sum = lax.dot_general(ohf, h2, (((0,), (0,)), ((), ())),
                                  preferred_element_type=jnp.float32)
    ones_h = jnp.ones((_BN, _H), jnp.float32)
    contrib_cnt = lax.dot_general(ohf, ones_h, (((0,), (0,)), ((), ())),
                                  preferred_element_type=jnp.float32)
    neg = jnp.float32(-jnp.inf)

    @pl.when(i == 0)
    def _():
        psum_ref[...] = jnp.zeros((_G, _H), jnp.float32)
        pcnt_ref[...] = jnp.zeros((_G, _H), jnp.float32)
        pmax_ref[...] = jnp.full((_G, _H), neg, jnp.float32)

    psum_ref[...] += contrib_sum
    pcnt_ref[...] += contrib_cnt

    glo = jnp.min(bt)
    ghi = jnp.max(bt)
    span = ghi - glo

    @pl.when(span < 16)
    def _():
        for w in range(16):
            gi = glo + w

            @pl.when(gi <= ghi)
            def _(gi=gi):
                e = (lax.broadcasted_iota(jnp.int32, (_G, 1), 0) ==
                     gi).astype(jnp.float32)
                mask = jnp.dot(ohf, e,
                               preferred_element_type=jnp.float32) > 0.5
                m = jnp.max(jnp.where(mask, h2, neg), axis=0, keepdims=True)
                cur = pmax_ref[pl.ds(gi, 1), :]
                pmax_ref[pl.ds(gi, 1), :] = jnp.maximum(cur, m)

    @pl.when(span >= 16)
    def _():
        for gidx in range(_G):
            @pl.when(jnp.logical_and(glo <= gidx, gidx <= ghi))
            def _(gidx=gidx):
                m = jnp.max(jnp.where(oh[:, gidx:gidx + 1], h2, neg), axis=0,
                            keepdims=True)
                pmax_ref[gidx:gidx + 1, :] = jnp.maximum(
                    pmax_ref[gidx:gidx + 1, :], m)


def _layer_pool(agg, cnt3, h, wl, wr, b, g, be, bt3):
    return pl.pallas_call(
        _layer_pool_body,
        grid=(_NBLK,),
        in_specs=[
            pl.BlockSpec((_NC, _BN, _H), lambda i: (0, i, 0)),
            pl.BlockSpec((_NC, _BN, _H), lambda i: (0, i, 0)),
            pl.BlockSpec((_BN, _H), lambda i: (i, 0)),
        ] + _W_SPECS + [
            pl.BlockSpec((1, 1, _BN), lambda i: (i, 0, 0)),
        ],
        out_specs=[
            pl.BlockSpec((_G, _H), lambda i: (0, 0)),
            pl.BlockSpec((_G, _H), lambda i: (0, 0)),
            pl.BlockSpec((_G, _H), lambda i: (0, 0)),
        ],
        out_shape=[
            jax.ShapeDtypeStruct((_G, _H), jnp.float32),
            jax.ShapeDtypeStruct((_G, _H), jnp.float32),
            jax.ShapeDtypeStruct((_G, _H), jnp.float32),
        ],
    )(agg, cnt3, h, wl, wr, b, g, be, bt3)


def _head_body(psum_ref, pmax_ref, pcnt_ref, w_ref, b_ref, out_ref):
    mean = psum_ref[...] / jnp.maximum(pcnt_ref[...], 1.0)
    pooled = jnp.concatenate([mean, pmax_ref[...]], axis=1)
    out_ref[...] = (jnp.dot(pooled, w_ref[...],
                            preferred_element_type=jnp.float32)
                    + b_ref[...])


def _head(psum, pmax, pcnt, linW, linb2):
    return pl.pallas_call(
        _head_body,
        out_shape=jax.ShapeDtypeStruct((_G, _OUT), jnp.float32),
    )(psum, pmax, pcnt, linW, linb2)


def kernel(x, xdims, edge_attr, edge_index, xsttype, batch, st_table,
           Wl0, Wr0, b0, g0, be0, Wl1, Wr1, b1, g1, be1, linW, linb):
    del edge_attr
    zrows = jnp.zeros((_N, _H), jnp.float32)
    xst3 = xsttype.reshape(_NBLK, 1, _BN)
    bt3 = batch.reshape(_NBLK, 1, _BN)

    src = edge_index[0]
    dst = edge_index[1]
    cnt3 = _sc_cached("cnt")(dst, zrows)
    h0 = _embed_concat(x, xdims, xst3, st_table)
    agg0 = _sc_cached("agg", _H, 80, 3)(h0, src, dst, zrows)
    h1 = _layer(agg0, cnt3, h0, Wl0, Wr0, b0, g0, be0)
    agg1 = _sc_cached("agg", _H, 80, 3)(h1, src, dst, zrows)
    psum, pmax, pcnt = _layer_pool(agg1, cnt3, h1, Wl1, Wr1, b1, g1, be1, bt3)
    return _head(psum, pmax, pcnt, linW, linb.reshape(1, _OUT))


# speedup vs baseline: 1.0288x; 1.0288x over previous
import functools

import jax
import jax.numpy as jnp
from jax import lax
from jax.experimental import pallas as pl
from jax.experimental.pallas import tpu as pltpu
from jax.experimental.pallas import tpu_sc as plsc

_N = 10000
_E = 320000
_G = 64
_DX = 104
_DD = 12
_EMB = 12
_NST = 256
_H = 128
_OUT = 10

_NC = 2
_NS = 16
_NW = _NC * _NS
_EPW = _E // _NW
_RPT = 624
_RREM = _N - _NS * _RPT
_NB = 3

_BN = 1000
_NBLK = _N // _BN


def _sc_agg_body(w, ch, nb, *refs):
    (h_hbm, src_hbm, dst_hbm, zrows_hbm, out_agg,
     acc, sidx, didx, *bufs) = refs
    rows = bufs[0]
    gsems = tuple(bufs[1:1 + nb])
    ssems = tuple(bufs[1 + nb:1 + 2 * nb])
    nch = _EPW // ch
    c = lax.axis_index("c")
    s = lax.axis_index("s")
    wid = s * _NC + c

    ebase = pl.multiple_of(wid * _EPW, 8)
    pltpu.sync_copy(src_hbm.at[pl.ds(ebase, _EPW)], sidx)
    pltpu.sync_copy(dst_hbm.at[pl.ds(ebase, _EPW)], didx)

    row0 = pl.multiple_of(s * _RPT, 8)
    pltpu.sync_copy(zrows_hbm.at[pl.ds(row0, _RPT)],
                    acc.at[pl.ds(row0, _RPT)])

    @pl.when(s == 0)
    def _():
        pltpu.sync_copy(zrows_hbm.at[pl.ds(_NS * _RPT, _RREM)],
                        acc.at[pl.ds(_NS * _RPT, _RREM)])
    plsc.subcore_barrier()

    def _idx(ref, j):
        st = pl.multiple_of(j * ch, 8)
        return ref.at[pl.ds(st, ch)]

    def _fire_g(j, b):
        pltpu.async_copy(h_hbm.at[_idx(sidx, j)], rows.at[b], gsems[b])

    def _wait_g(b):
        pltpu.make_async_copy(h_hbm.at[_idx(sidx, 0)], rows.at[b],
                              gsems[b]).wait()

    def _fire_s(j, b):
        pltpu.async_copy(rows.at[b], acc.at[_idx(didx, j)], ssems[b],
                         add=True)

    def _wait_s(b):
        pltpu.make_async_copy(rows.at[b], acc.at[_idx(didx, 0)],
                              ssems[b]).wait()

    lag = nb - 1
    for b in range(nb):
        _fire_g(b, b)
    for k in range(lag):
        _wait_g(k % nb)
        _fire_s(k, k % nb)
        _wait_s(k % nb)
        _fire_g(k + nb, k % nb)
    for k in range(lag, nb):
        _wait_g(k)
        _fire_s(k, k)

    ngrp = (nch - lag) // nb
    def _group(grp, carry):
        for r in range(nb):
            k = grp * nb + r
            ba = (r + lag) % nb
            _wait_s(ba)
            _fire_g(k + lag, ba)
            _wait_g(r)
            _fire_s(k, r)
        return carry

    lax.fori_loop(1, ngrp, _group, 0)

    for j in range(ngrp * nb, nch):
        if j + lag < nch:
            _wait_s((j + lag) % nb)
            _fire_g(j + lag, (j + lag) % nb)
        _wait_g(j % nb)
        _fire_s(j, j % nb)
    for b in range(nb):
        _wait_s(b)
    plsc.subcore_barrier()

    pltpu.sync_copy(acc.at[pl.ds(row0, _RPT)],
                    out_agg.at[c, pl.ds(row0, _RPT)])

    @pl.when(s == 0)
    def _():
        pltpu.sync_copy(acc.at[pl.ds(_NS * _RPT, _RREM)],
                        out_agg.at[c, pl.ds(_NS * _RPT, _RREM)])


def _sc_mesh():
    return plsc.VectorSubcoreMesh(core_axis_name="c", subcore_axis_name="s",
                                  num_cores=_NC, num_subcores=_NS)


def _make_sc_agg(w, ch, nb):
    return pl.kernel(
        functools.partial(_sc_agg_body, w, ch, nb),
        out_type=jax.ShapeDtypeStruct((_NC, _N, w), jnp.float32),
        mesh=_sc_mesh(),
        scratch_types=[
            pltpu.VMEM_SHARED((_N, w), jnp.float32),
            pltpu.VMEM((_EPW,), jnp.int32),
            pltpu.VMEM((_EPW,), jnp.int32),
            pltpu.VMEM((nb, ch, w), jnp.float32),
        ] + [pltpu.SemaphoreType.DMA] * (2 * nb),
    )


def _sc_cnt_body(dst_hbm, zrows_hbm, out_cnt, cacc, didx, ones, csem):
    c = lax.axis_index("c")
    s = lax.axis_index("s")
    wid = s * _NC + c

    ebase = pl.multiple_of(wid * _EPW, 8)
    pltpu.sync_copy(dst_hbm.at[pl.ds(ebase, _EPW)], didx)
    row0 = pl.multiple_of(s * _RPT, 8)
    pltpu.sync_copy(zrows_hbm.at[pl.ds(row0, _RPT)],
                    cacc.at[pl.ds(row0, _RPT)])

    @pl.when(s == 0)
    def _():
        pltpu.sync_copy(zrows_hbm.at[pl.ds(_NS * _RPT, _RREM)],
                        cacc.at[pl.ds(_NS * _RPT, _RREM)])

    def _fill(k, carry):
        for q in range(_H // 16):
            ones[k, pl.ds(q * 16, 16)] = jnp.full((16,), 1.0, jnp.float32)
        return carry

    lax.fori_loop(0, 80, _fill, 0)
    plsc.subcore_barrier()

    def _scat(j):
        st = pl.multiple_of(j * 80, 8)
        pltpu.async_copy(ones, cacc.at[didx.at[pl.ds(st, 80)]], csem,
                         add=True)

    def _drain():
        pltpu.make_async_copy(ones, cacc.at[didx.at[pl.ds(0, 80)]],
                              csem).wait()

    grp = 5
    def _body(g, carry):
        for b in range(grp):
            _scat(g * grp + b)
        for _ in range(grp):
            _drain()
        return carry

    lax.fori_loop(0, (_EPW // 80) // grp, _body, 0)
    plsc.subcore_barrier()

    pltpu.sync_copy(cacc.at[pl.ds(row0, _RPT)],
                    out_cnt.at[c, pl.ds(row0, _RPT)])

    @pl.when(s == 0)
    def _():
        pltpu.sync_copy(cacc.at[pl.ds(_NS * _RPT, _RREM)],
                        out_cnt.at[c, pl.ds(_NS * _RPT, _RREM)])


def _make_sc_cnt():
    return pl.kernel(
        _sc_cnt_body,
        out_type=jax.ShapeDtypeStruct((_NC, _N, _H), jnp.float32),
        mesh=_sc_mesh(),
        scratch_types=[
            pltpu.VMEM_SHARED((_N, _H), jnp.float32),
            pltpu.VMEM((_EPW,), jnp.int32),
            pltpu.VMEM((80, _H), jnp.float32),
            pltpu.SemaphoreType.DMA,
        ],
    )


@functools.lru_cache(maxsize=None)
def _sc_cached(kind, *args):
    return _make_sc_agg(*args) if kind == "agg" else _make_sc_cnt()


def _embed_body(x_ref, xd_ref, xst_ref, st_ref, out_ref):
    xst = xst_ref[0, 0, :]
    oh = (xst[:, None] == lax.broadcasted_iota(jnp.int32, (_BN, _NST), 1))
    emb = jnp.dot(oh.astype(jnp.float32), st_ref[...],
                  preferred_element_type=jnp.float32)
    out_ref[...] = jnp.concatenate([x_ref[...], xd_ref[...], emb], axis=1)


def _embed_concat(x, xdims, xst3, st_table):
    return pl.pallas_call(
        _embed_body,
        grid=(_NBLK,),
        in_specs=[
            pl.BlockSpec((_BN, _DX), lambda i: (i, 0)),
            pl.BlockSpec((_BN, _DD), lambda i: (i, 0)),
            pl.BlockSpec((1, 1, _BN), lambda i: (i, 0, 0)),
            pl.BlockSpec((_NST, _EMB), lambda i: (0, 0)),
        ],
        out_specs=pl.BlockSpec((_BN, _H), lambda i: (i, 0)),
        out_shape=jax.ShapeDtypeStruct((_N, _H), jnp.float32),
    )(x, xdims, xst3, st_table)


def _layer_math(agg_ref, cnt_ref, h_ref, wl_ref, wr_ref, b_ref, g_ref, be_ref):
    cnt = cnt_ref[0, :, 0:1] + cnt_ref[1, :, 0:1]
    inv = 1.0 / jnp.maximum(cnt, 1.0)
    mean = (agg_ref[0] + agg_ref[1]) * inv
    z = (jnp.dot(mean, wl_ref[...], preferred_element_type=jnp.float32)
         + jnp.dot(h_ref[...], wr_ref[...], preferred_element_type=jnp.float32)
         + b_ref[...])
    m = jnp.mean(z, axis=-1, keepdims=True)
    zc = z - m
    v = jnp.mean(zc * zc, axis=-1, keepdims=True)
    hn = zc * lax.rsqrt(v + 1e-5) * g_ref[...] + be_ref[...]
    return jnp.maximum(hn, 0.0)


def _layer_body(agg_ref, cnt_ref, h_ref, wl_ref, wr_ref, b_ref, g_ref,
                be_ref, out_ref):
    out_ref[...] = _layer_math(agg_ref, cnt_ref, h_ref, wl_ref, wr_ref,
                               b_ref, g_ref, be_ref)


_W_SPECS = [
    pl.BlockSpec((_H, _H), lambda i: (0, 0)),
    pl.BlockSpec((_H, _H), lambda i: (0, 0)),
    pl.BlockSpec((_H,), lambda i: (0,)),
    pl.BlockSpec((_H,), lambda i: (0,)),
    pl.BlockSpec((_H,), lambda i: (0,)),
]


def _layer(agg, cnt3, h, wl, wr, b, g, be):
    return pl.pallas_call(
        _layer_body,
        grid=(_NBLK,),
        in_specs=[
            pl.BlockSpec((_NC, _BN, _H), lambda i: (0, i, 0)),
            pl.BlockSpec((_NC, _BN, 8), lambda i: (0, i, 0)),
            pl.BlockSpec((_BN, _H), lambda i: (i, 0)),
        ] + _W_SPECS,
        out_specs=pl.BlockSpec((_BN, _H), lambda i: (i, 0)),
        out_shape=jax.ShapeDtypeStruct((_N, _H), jnp.float32),
    )(agg, cnt3, h, wl, wr, b, g, be)


def _layer_pool_body(agg_ref, cnt_ref, h_ref, wl_ref, wr_ref, b_ref, g_ref,
                     be_ref, bt_ref, btc_ref, psum_ref, pmax_ref, pcnt_ref):
    i = pl.program_id(0)
    h2 = _layer_math(agg_ref, cnt_ref, h_ref, wl_ref, wr_ref, b_ref, g_ref,
                     be_ref)
    bt = bt_ref[0, 0, :]
    oh = (bt[:, None] == lax.broadcasted_iota(jnp.int32, (_BN, _G), 1))
    ohf = oh.astype(jnp.float32)
    contrib_sum = lax.dot_general(ohf, h2, (((0,), (0,)), ((), ())),
                                  preferred_element_type=jnp.float32)
    ones_h = jnp.ones((_BN, _H), jnp.float32)
    contrib_cnt = lax.dot_general(ohf, ones_h, (((0,), (0,)), ((), ())),
                                  preferred_element_type=jnp.float32)
    neg = jnp.float32(-jnp.inf)

    @pl.when(i == 0)
    def _():
        psum_ref[...] = jnp.zeros((_G, _H), jnp.float32)
        pcnt_ref[...] = jnp.zeros((_G, _H), jnp.float32)
        pmax_ref[...] = jnp.full((_G, _H), neg, jnp.float32)

    psum_ref[...] += contrib_sum
    pcnt_ref[...] += contrib_cnt

    glo = jnp.min(bt)
    ghi = jnp.max(bt)
    span = ghi - glo

    @pl.when(span < 16)
    def _():
        for w in range(16):
            gi = glo + w

            @pl.when(gi <= ghi)
            def _(gi=gi):
                mask = btc_ref[...] == gi
                m = jnp.max(jnp.where(mask, h2, neg), axis=0, keepdims=True)
                cur = pmax_ref[pl.ds(gi, 1), :]
                pmax_ref[pl.ds(gi, 1), :] = jnp.maximum(cur, m)

    @pl.when(span >= 16)
    def _():
        for gidx in range(_G):
            @pl.when(jnp.logical_and(glo <= gidx, gidx <= ghi))
            def _(gidx=gidx):
                m = jnp.max(jnp.where(oh[:, gidx:gidx + 1], h2, neg), axis=0,
                            keepdims=True)
                pmax_ref[gidx:gidx + 1, :] = jnp.maximum(
                    pmax_ref[gidx:gidx + 1, :], m)


def _layer_pool(agg, cnt3, h, wl, wr, b, g, be, bt3, btc):
    return pl.pallas_call(
        _layer_pool_body,
        grid=(_NBLK,),
        in_specs=[
            pl.BlockSpec((_NC, _BN, _H), lambda i: (0, i, 0)),
            pl.BlockSpec((_NC, _BN, 8), lambda i: (0, i, 0)),
            pl.BlockSpec((_BN, _H), lambda i: (i, 0)),
        ] + _W_SPECS + [
            pl.BlockSpec((1, 1, _BN), lambda i: (i, 0, 0)),
            pl.BlockSpec((_BN, 1), lambda i: (i, 0)),
        ],
        out_specs=[
            pl.BlockSpec((_G, _H), lambda i: (0, 0)),
            pl.BlockSpec((_G, _H), lambda i: (0, 0)),
            pl.BlockSpec((_G, _H), lambda i: (0, 0)),
        ],
        out_shape=[
            jax.ShapeDtypeStruct((_G, _H), jnp.float32),
            jax.ShapeDtypeStruct((_G, _H), jnp.float32),
            jax.ShapeDtypeStruct((_G, _H), jnp.float32),
        ],
    )(agg, cnt3, h, wl, wr, b, g, be, bt3, btc)


def _head_body(psum_ref, pmax_ref, pcnt_ref, w_ref, b_ref, out_ref):
    mean = psum_ref[...] / jnp.maximum(pcnt_ref[...], 1.0)
    pooled = jnp.concatenate([mean, pmax_ref[...]], axis=1)
    out_ref[...] = (jnp.dot(pooled, w_ref[...],
                            preferred_element_type=jnp.float32)
                    + b_ref[...])


def _head(psum, pmax, pcnt, linW, linb2):
    return pl.pallas_call(
        _head_body,
        out_shape=jax.ShapeDtypeStruct((_G, _OUT), jnp.float32),
    )(psum, pmax, pcnt, linW, linb2)


def kernel(x, xdims, edge_attr, edge_index, xsttype, batch, st_table,
           Wl0, Wr0, b0, g0, be0, Wl1, Wr1, b1, g1, be1, linW, linb):
    del edge_attr
    zrows = jnp.zeros((_N, _H), jnp.float32)
    xst3 = xsttype.reshape(_NBLK, 1, _BN)
    bt3 = batch.reshape(_NBLK, 1, _BN)
    btc = batch.reshape(_N, 1)

    src = edge_index[0]
    dst = edge_index[1]
    cnt3 = _sc_cached("cnt")(dst, zrows)[:, :, 0:8]
    h0 = _embed_concat(x, xdims, xst3, st_table)
    agg0 = _sc_cached("agg", _H, 80, 3)(h0, src, dst, zrows)
    h1 = _layer(agg0, cnt3, h0, Wl0, Wr0, b0, g0, be0)
    agg1 = _sc_cached("agg", _H, 80, 3)(h1, src, dst, zrows)
    psum, pmax, pcnt = _layer_pool(agg1, cnt3, h1, Wl1, Wr1, b1, g1, be1,
                                   bt3, btc)
    return _head(psum, pmax, pcnt, linW, linb.reshape(1, _OUT))


# speedup vs baseline: 1.0461x; 1.0168x over previous
import functools

import jax
import jax.numpy as jnp
from jax import lax
from jax.experimental import pallas as pl
from jax.experimental.pallas import tpu as pltpu
from jax.experimental.pallas import tpu_sc as plsc

_N = 10000
_E = 320000
_G = 64
_DX = 104
_DD = 12
_EMB = 12
_NST = 256
_H = 128
_OUT = 10

_NC = 2
_NS = 16
_NW = _NC * _NS
_EPW = _E // _NW
_RPT = 624
_RREM = _N - _NS * _RPT
_NB = 3

_BN = 2000
_NBLK = _N // _BN


def _sc_agg_body(w, ch, nb, *refs):
    (h_hbm, src_hbm, dst_hbm, zrows_hbm, out_agg,
     acc, sidx, didx, *bufs) = refs
    rows = bufs[0]
    gsems = tuple(bufs[1:1 + nb])
    ssems = tuple(bufs[1 + nb:1 + 2 * nb])
    nch = _EPW // ch
    c = lax.axis_index("c")
    s = lax.axis_index("s")
    wid = s * _NC + c

    ebase = pl.multiple_of(wid * _EPW, 8)
    pltpu.sync_copy(src_hbm.at[pl.ds(ebase, _EPW)], sidx)
    pltpu.sync_copy(dst_hbm.at[pl.ds(ebase, _EPW)], didx)

    row0 = pl.multiple_of(s * _RPT, 8)
    pltpu.sync_copy(zrows_hbm.at[pl.ds(row0, _RPT)],
                    acc.at[pl.ds(row0, _RPT)])

    @pl.when(s == 0)
    def _():
        pltpu.sync_copy(zrows_hbm.at[pl.ds(_NS * _RPT, _RREM)],
                        acc.at[pl.ds(_NS * _RPT, _RREM)])
    plsc.subcore_barrier()

    def _idx(ref, j):
        st = pl.multiple_of(j * ch, 8)
        return ref.at[pl.ds(st, ch)]

    def _fire_g(j, b):
        pltpu.async_copy(h_hbm.at[_idx(sidx, j)], rows.at[b], gsems[b])

    def _wait_g(b):
        pltpu.make_async_copy(h_hbm.at[_idx(sidx, 0)], rows.at[b],
                              gsems[b]).wait()

    def _fire_s(j, b):
        pltpu.async_copy(rows.at[b], acc.at[_idx(didx, j)], ssems[b],
                         add=True)

    def _wait_s(b):
        pltpu.make_async_copy(rows.at[b], acc.at[_idx(didx, 0)],
                              ssems[b]).wait()

    lag = nb - 1
    for b in range(nb):
        _fire_g(b, b)
    for k in range(lag):
        _wait_g(k % nb)
        _fire_s(k, k % nb)
        _wait_s(k % nb)
        _fire_g(k + nb, k % nb)
    for k in range(lag, nb):
        _wait_g(k)
        _fire_s(k, k)

    ngrp = (nch - lag) // nb
    def _group(grp, carry):
        for r in range(nb):
            k = grp * nb + r
            ba = (r + lag) % nb
            _wait_s(ba)
            _fire_g(k + lag, ba)
            _wait_g(r)
            _fire_s(k, r)
        return carry

    lax.fori_loop(1, ngrp, _group, 0)

    for j in range(ngrp * nb, nch):
        if j + lag < nch:
            _wait_s((j + lag) % nb)
            _fire_g(j + lag, (j + lag) % nb)
        _wait_g(j % nb)
        _fire_s(j, j % nb)
    for b in range(nb):
        _wait_s(b)
    plsc.subcore_barrier()

    pltpu.sync_copy(acc.at[pl.ds(row0, _RPT)],
                    out_agg.at[c, pl.ds(row0, _RPT)])

    @pl.when(s == 0)
    def _():
        pltpu.sync_copy(acc.at[pl.ds(_NS * _RPT, _RREM)],
                        out_agg.at[c, pl.ds(_NS * _RPT, _RREM)])


def _sc_mesh():
    return plsc.VectorSubcoreMesh(core_axis_name="c", subcore_axis_name="s",
                                  num_cores=_NC, num_subcores=_NS)


def _make_sc_agg(w, ch, nb):
    return pl.kernel(
        functools.partial(_sc_agg_body, w, ch, nb),
        out_type=jax.ShapeDtypeStruct((_NC, _N, w), jnp.float32),
        mesh=_sc_mesh(),
        scratch_types=[
            pltpu.VMEM_SHARED((_N, w), jnp.float32),
            pltpu.VMEM((_EPW,), jnp.int32),
            pltpu.VMEM((_EPW,), jnp.int32),
            pltpu.VMEM((nb, ch, w), jnp.float32),
        ] + [pltpu.SemaphoreType.DMA] * (2 * nb),
    )


def _sc_cnt_body(dst_hbm, zrows_hbm, out_cnt, cacc, didx, ones, csem):
    c = lax.axis_index("c")
    s = lax.axis_index("s")
    wid = s * _NC + c

    ebase = pl.multiple_of(wid * _EPW, 8)
    pltpu.sync_copy(dst_hbm.at[pl.ds(ebase, _EPW)], didx)
    row0 = pl.multiple_of(s * _RPT, 8)
    pltpu.sync_copy(zrows_hbm.at[pl.ds(row0, _RPT)],
                    cacc.at[pl.ds(row0, _RPT)])

    @pl.when(s == 0)
    def _():
        pltpu.sync_copy(zrows_hbm.at[pl.ds(_NS * _RPT, _RREM)],
                        cacc.at[pl.ds(_NS * _RPT, _RREM)])

    def _fill(k, carry):
        for q in range(_H // 16):
            ones[k, pl.ds(q * 16, 16)] = jnp.full((16,), 1.0, jnp.float32)
        return carry

    lax.fori_loop(0, 80, _fill, 0)
    plsc.subcore_barrier()

    def _scat(j):
        st = pl.multiple_of(j * 80, 8)
        pltpu.async_copy(ones, cacc.at[didx.at[pl.ds(st, 80)]], csem,
                         add=True)

    def _drain():
        pltpu.make_async_copy(ones, cacc.at[didx.at[pl.ds(0, 80)]],
                              csem).wait()

    grp = 5
    def _body(g, carry):
        for b in range(grp):
            _scat(g * grp + b)
        for _ in range(grp):
            _drain()
        return carry

    lax.fori_loop(0, (_EPW // 80) // grp, _body, 0)
    plsc.subcore_barrier()

    pltpu.sync_copy(cacc.at[pl.ds(row0, _RPT)],
                    out_cnt.at[c, pl.ds(row0, _RPT)])

    @pl.when(s == 0)
    def _():
        pltpu.sync_copy(cacc.at[pl.ds(_NS * _RPT, _RREM)],
                        out_cnt.at[c, pl.ds(_NS * _RPT, _RREM)])


def _make_sc_cnt():
    return pl.kernel(
        _sc_cnt_body,
        out_type=jax.ShapeDtypeStruct((_NC, _N, _H), jnp.float32),
        mesh=_sc_mesh(),
        scratch_types=[
            pltpu.VMEM_SHARED((_N, _H), jnp.float32),
            pltpu.VMEM((_EPW,), jnp.int32),
            pltpu.VMEM((80, _H), jnp.float32),
            pltpu.SemaphoreType.DMA,
        ],
    )


@functools.lru_cache(maxsize=None)
def _sc_cached(kind, *args):
    return _make_sc_agg(*args) if kind == "agg" else _make_sc_cnt()


def _embed_body(x_ref, xd_ref, xst_ref, st_ref, out_ref):
    xst = xst_ref[0, 0, :]
    oh = (xst[:, None] == lax.broadcasted_iota(jnp.int32, (_BN, _NST), 1))
    emb = jnp.dot(oh.astype(jnp.float32), st_ref[...],
                  preferred_element_type=jnp.float32)
    out_ref[...] = jnp.concatenate([x_ref[...], xd_ref[...], emb], axis=1)


def _embed_concat(x, xdims, xst3, st_table):
    return pl.pallas_call(
        _embed_body,
        grid=(_NBLK,),
        in_specs=[
            pl.BlockSpec((_BN, _DX), lambda i: (i, 0)),
            pl.BlockSpec((_BN, _DD), lambda i: (i, 0)),
            pl.BlockSpec((1, 1, _BN), lambda i: (i, 0, 0)),
            pl.BlockSpec((_NST, _EMB), lambda i: (0, 0)),
        ],
        out_specs=pl.BlockSpec((_BN, _H), lambda i: (i, 0)),
        out_shape=jax.ShapeDtypeStruct((_N, _H), jnp.float32),
    )(x, xdims, xst3, st_table)


def _layer_math(agg_ref, cnt_ref, h_ref, wl_ref, wr_ref, b_ref, g_ref, be_ref):
    cnt = cnt_ref[0, :, 0:1] + cnt_ref[1, :, 0:1]
    inv = 1.0 / jnp.maximum(cnt, 1.0)
    mean = (agg_ref[0] + agg_ref[1]) * inv
    z = (jnp.dot(mean, wl_ref[...], preferred_element_type=jnp.float32)
         + jnp.dot(h_ref[...], wr_ref[...], preferred_element_type=jnp.float32)
         + b_ref[...])
    m = jnp.mean(z, axis=-1, keepdims=True)
    zc = z - m
    v = jnp.mean(zc * zc, axis=-1, keepdims=True)
    hn = zc * lax.rsqrt(v + 1e-5) * g_ref[...] + be_ref[...]
    return jnp.maximum(hn, 0.0)


def _layer_body(agg_ref, cnt_ref, h_ref, wl_ref, wr_ref, b_ref, g_ref,
                be_ref, out_ref):
    out_ref[...] = _layer_math(agg_ref, cnt_ref, h_ref, wl_ref, wr_ref,
                               b_ref, g_ref, be_ref)


_W_SPECS = [
    pl.BlockSpec((_H, _H), lambda i: (0, 0)),
    pl.BlockSpec((_H, _H), lambda i: (0, 0)),
    pl.BlockSpec((_H,), lambda i: (0,)),
    pl.BlockSpec((_H,), lambda i: (0,)),
    pl.BlockSpec((_H,), lambda i: (0,)),
]


def _layer(agg, cnt3, h, wl, wr, b, g, be):
    return pl.pallas_call(
        _layer_body,
        grid=(_NBLK,),
        in_specs=[
            pl.BlockSpec((_NC, _BN, _H), lambda i: (0, i, 0)),
            pl.BlockSpec((_NC, _BN, 8), lambda i: (0, i, 0)),
            pl.BlockSpec((_BN, _H), lambda i: (i, 0)),
        ] + _W_SPECS,
        out_specs=pl.BlockSpec((_BN, _H), lambda i: (i, 0)),
        out_shape=jax.ShapeDtypeStruct((_N, _H), jnp.float32),
    )(agg, cnt3, h, wl, wr, b, g, be)


def _layer_pool_body(agg_ref, cnt_ref, h_ref, wl_ref, wr_ref, b_ref, g_ref,
                     be_ref, bt_ref, btc_ref, psum_ref, pmax_ref, pcnt_ref):
    i = pl.program_id(0)
    h2 = _layer_math(agg_ref, cnt_ref, h_ref, wl_ref, wr_ref, b_ref, g_ref,
                     be_ref)
    bt = bt_ref[0, 0, :]
    oh = (bt[:, None] == lax.broadcasted_iota(jnp.int32, (_BN, _G), 1))
    ohf = oh.astype(jnp.float32)
    contrib_sum = lax.dot_general(ohf, h2, (((0,), (0,)), ((), ())),
                                  preferred_element_type=jnp.float32)
    ones_h = jnp.ones((_BN, _H), jnp.float32)
    contrib_cnt = lax.dot_general(ohf, ones_h, (((0,), (0,)), ((), ())),
                                  preferred_element_type=jnp.float32)
    neg = jnp.float32(-jnp.inf)

    @pl.when(i == 0)
    def _():
        psum_ref[...] = jnp.zeros((_G, _H), jnp.float32)
        pcnt_ref[...] = jnp.zeros((_G, _H), jnp.float32)
        pmax_ref[...] = jnp.full((_G, _H), neg, jnp.float32)

    psum_ref[...] += contrib_sum
    pcnt_ref[...] += contrib_cnt

    glo = jnp.min(bt)
    ghi = jnp.max(bt)
    span = ghi - glo

    @pl.when(span < 24)
    def _():
        for w in range(24):
            gi = glo + w

            @pl.when(gi <= ghi)
            def _(gi=gi):
                mask = btc_ref[...] == gi
                m = jnp.max(jnp.where(mask, h2, neg), axis=0, keepdims=True)
                cur = pmax_ref[pl.ds(gi, 1), :]
                pmax_ref[pl.ds(gi, 1), :] = jnp.maximum(cur, m)

    @pl.when(span >= 24)
    def _():
        for gidx in range(_G):
            @pl.when(jnp.logical_and(glo <= gidx, gidx <= ghi))
            def _(gidx=gidx):
                m = jnp.max(jnp.where(oh[:, gidx:gidx + 1], h2, neg), axis=0,
                            keepdims=True)
                pmax_ref[gidx:gidx + 1, :] = jnp.maximum(
                    pmax_ref[gidx:gidx + 1, :], m)


def _layer_pool(agg, cnt3, h, wl, wr, b, g, be, bt3, btc):
    return pl.pallas_call(
        _layer_pool_body,
        grid=(_NBLK,),
        in_specs=[
            pl.BlockSpec((_NC, _BN, _H), lambda i: (0, i, 0)),
            pl.BlockSpec((_NC, _BN, 8), lambda i: (0, i, 0)),
            pl.BlockSpec((_BN, _H), lambda i: (i, 0)),
        ] + _W_SPECS + [
            pl.BlockSpec((1, 1, _BN), lambda i: (i, 0, 0)),
            pl.BlockSpec((_BN, 1), lambda i: (i, 0)),
        ],
        out_specs=[
            pl.BlockSpec((_G, _H), lambda i: (0, 0)),
            pl.BlockSpec((_G, _H), lambda i: (0, 0)),
            pl.BlockSpec((_G, _H), lambda i: (0, 0)),
        ],
        out_shape=[
            jax.ShapeDtypeStruct((_G, _H), jnp.float32),
            jax.ShapeDtypeStruct((_G, _H), jnp.float32),
            jax.ShapeDtypeStruct((_G, _H), jnp.float32),
        ],
    )(agg, cnt3, h, wl, wr, b, g, be, bt3, btc)


def _head_body(psum_ref, pmax_ref, pcnt_ref, w_ref, b_ref, out_ref):
    mean = psum_ref[...] / jnp.maximum(pcnt_ref[...], 1.0)
    pooled = jnp.concatenate([mean, pmax_ref[...]], axis=1)
    out_ref[...] = (jnp.dot(pooled, w_ref[...],
                            preferred_element_type=jnp.float32)
                    + b_ref[...])


def _head(psum, pmax, pcnt, linW, linb2):
    return pl.pallas_call(
        _head_body,
        out_shape=jax.ShapeDtypeStruct((_G, _OUT), jnp.float32),
    )(psum, pmax, pcnt, linW, linb2)


def kernel(x, xdims, edge_attr, edge_index, xsttype, batch, st_table,
           Wl0, Wr0, b0, g0, be0, Wl1, Wr1, b1, g1, be1, linW, linb):
    del edge_attr
    zrows = jnp.zeros((_N, _H), jnp.float32)
    xst3 = xsttype.reshape(_NBLK, 1, _BN)
    bt3 = batch.reshape(_NBLK, 1, _BN)
    btc = batch.reshape(_N, 1)

    src = edge_index[0]
    dst = edge_index[1]
    cnt3 = _sc_cached("cnt")(dst, zrows)[:, :, 0:8]
    h0 = _embed_concat(x, xdims, xst3, st_table)
    agg0 = _sc_cached("agg", _H, 80, 3)(h0, src, dst, zrows)
    h1 = _layer(agg0, cnt3, h0, Wl0, Wr0, b0, g0, be0)
    agg1 = _sc_cached("agg", _H, 80, 3)(h1, src, dst, zrows)
    psum, pmax, pcnt = _layer_pool(agg1, cnt3, h1, Wl1, Wr1, b1, g1, be1,
                                   bt3, btc)
    return _head(psum, pmax, pcnt, linW, linb.reshape(1, _OUT))
